# baseline (speedup 1.0000x reference)
import functools

import jax
import jax.numpy as jnp
from jax import lax
from jax.experimental import pallas as pl
from jax.experimental.pallas import tpu as pltpu
from jax.experimental.pallas import tpu_sc as plsc

N_NODES = 10000
D = 256
DH = 128
EPS = 1e-5
SLOPE = 0.01
NSUB = 16
CHUNK = 128
N_ACC = 10112
ROWS_PER_TILE = N_ACC // NSUB
DEG_SLAB = 640
DEG_PAD = NSUB * DEG_SLAB

_MESH = plsc.VectorSubcoreMesh(core_axis_name="c", subcore_axis_name="s")


def _deg_body(n_ch, dst_hbm, ew_hbm, out_hbm, didx_v, ew_v, zero_v, acc_sh,
              sem):
    del sem
    c = lax.axis_index("c")
    s = lax.axis_index("s")
    for k in range(DEG_SLAB // 16):
        zero_v[pl.ds(k * 16, 16)] = jnp.zeros((16,), jnp.float32)
    pltpu.sync_copy(zero_v, acc_sh.at[pl.ds(s * DEG_SLAB, DEG_SLAB)])
    plsc.subcore_barrier()
    half = (n_ch + 1) // 2
    lo = jnp.where(c == 0, 0, half)
    hi = jnp.where(c == 0, half, n_ch)

    def body(j, carry):
        off = pl.multiple_of((s * n_ch + j) * CHUNK, CHUNK)
        pltpu.sync_copy(dst_hbm.at[pl.ds(off, CHUNK)], didx_v)
        pltpu.sync_copy(ew_hbm.at[pl.ds(off, CHUNK)], ew_v)
        pltpu.sync_copy(ew_v, acc_sh.at[didx_v], add=True)
        return carry

    lax.fori_loop(lo, hi, body, 0)
    plsc.subcore_barrier()
    out_off = pl.multiple_of(c * DEG_PAD + s * DEG_SLAB, DEG_SLAB)
    pltpu.sync_copy(acc_sh.at[pl.ds(s * DEG_SLAB, DEG_SLAB)],
                    out_hbm.at[pl.ds(out_off, DEG_SLAB)])


def _make_deg_call(n_ch):
    return functools.partial(
        pl.kernel,
        mesh=_MESH,
        out_type=jax.ShapeDtypeStruct((2 * DEG_PAD,), jnp.float32),
        scratch_types=[
            pltpu.VMEM((CHUNK,), jnp.int32),
            pltpu.VMEM((CHUNK,), jnp.float32),
            pltpu.VMEM((DEG_SLAB,), jnp.float32),
            pltpu.VMEM_SHARED((DEG_PAD,), jnp.float32),
            pltpu.SemaphoreType.DMA,
        ],
    )(functools.partial(_deg_body, n_ch))


def _agg_body(n_ch, y_hbm, src_hbm, dst_hbm, ew_hbm, out_hbm,
              dst_all, sidx0_v, sidx1_v, ew0_v, ew1_v, rows0_v, rows1_v,
              acc_sh, semg0, semg1, semw0, semw1, sems0, sems1):
    c = lax.axis_index("c")
    s = lax.axis_index("s")
    y_base = c * N_ACC
    row0 = s * ROWS_PER_TILE
    slab0 = pl.multiple_of(y_base + row0, 8)
    rows = (rows0_v, rows1_v)
    ews = (ew0_v, ew1_v)
    sidx = (sidx0_v, sidx1_v)
    semg = (semg0, semg1)
    semw = (semw0, semw1)
    sems = (sems0, sems1)
    sbase = s * n_ch * CHUNK
    cbase = c * (NSUB * n_ch * CHUNK) + sbase
    n_pairs = n_ch // 2
    _dn = lax.GatherDimensionNumbers(offset_dims=(), collapsed_slice_dims=(0,),
                                     start_index_map=(0,))

    pltpu.sync_copy(dst_hbm.at[s], dst_all)
    pltpu.sync_copy(y_hbm.at[pl.ds(slab0, ROWS_PER_TILE)],
                    acc_sh.at[pl.ds(row0, ROWS_PER_TILE)])
    plsc.subcore_barrier()

    def loads(j, slot):
        coff = pl.multiple_of(cbase + j * CHUNK, CHUNK)
        pltpu.make_async_copy(src_hbm.at[pl.ds(coff, CHUNK)],
                              sidx[slot], sems[slot]).start()
        soff = pl.multiple_of(sbase + j * CHUNK, CHUNK)
        pltpu.make_async_copy(ew_hbm.at[pl.ds(soff, CHUNK)],
                              ews[slot], semw[slot]).start()

    def gather(j, slot):
        soff = pl.multiple_of(cbase + j * CHUNK, CHUNK)
        pltpu.make_async_copy(src_hbm.at[pl.ds(soff, CHUNK)],
                              sidx[slot], sems[slot]).wait()
        pltpu.make_async_copy(y_hbm.at[sidx[slot]], rows[slot],
                              semg[slot]).start()

    def consume(j, slot):
        pltpu.make_async_copy(y_hbm.at[sidx[slot]], rows[slot],
                              semg[slot]).wait()
        soff = pl.multiple_of(sbase + j * CHUNK, CHUNK)
        pltpu.make_async_copy(ew_hbm.at[pl.ds(soff, CHUNK)],
                              ews[slot], semw[slot]).wait()

        def grp(g, inner):
            g16 = pl.multiple_of(g * 16, 16)
            ew16 = ews[slot][pl.ds(g16, 16)]
            for l in range(16):
                w16 = lax.gather(ew16, jnp.full((16, 1), l, jnp.int32), _dn,
                                 slice_sizes=(1,),
                                 mode=lax.GatherScatterMode.PROMISE_IN_BOUNDS)
                e = g16 + l
                for k in range(DH // 16):
                    rows[slot][e, pl.ds(k * 16, 16)] = (
                        rows[slot][e, pl.ds(k * 16, 16)] * w16)
            return inner

        lax.fori_loop(0, CHUNK // 16, grp, 0)
        pltpu.sync_copy(rows[slot], acc_sh.at[dst_all.at[j]], add=True)

    loads(0, 0)
    gather(0, 0)
    loads(1, 1)

    def body(p, carry):
        j0 = 2 * p
        j1 = j0 + 1
        more = p + 1 < n_pairs
        gather(j1, 1)
        consume(j0, 0)

        @pl.when(more)
        def _n0():
            loads(j0 + 2, 0)
            gather(j0 + 2, 0)

        consume(j1, 1)

        @pl.when(more)
        def _n1():
            loads(j1 + 2, 1)

        return carry

    lax.fori_loop(0, n_pairs, body, 0)
    plsc.subcore_barrier()
    pltpu.sync_copy(acc_sh.at[pl.ds(row0, ROWS_PER_TILE)],
                    out_hbm.at[pl.ds(slab0, ROWS_PER_TILE)])


def _make_agg_call(n_ch):
    return functools.partial(
        pl.kernel,
        mesh=_MESH,
        out_type=jax.ShapeDtypeStruct((2 * N_ACC, DH), jnp.float32),
        scratch_types=[
            pltpu.VMEM((n_ch, CHUNK), jnp.int32),
            pltpu.VMEM((CHUNK,), jnp.int32),
            pltpu.VMEM((CHUNK,), jnp.int32),
            pltpu.VMEM((CHUNK,), jnp.float32),
            pltpu.VMEM((CHUNK,), jnp.float32),
            pltpu.VMEM((CHUNK, DH), jnp.float32),
            pltpu.VMEM((CHUNK, DH), jnp.float32),
            pltpu.VMEM_SHARED((N_ACC, DH), jnp.float32),
            pltpu.SemaphoreType.DMA,
            pltpu.SemaphoreType.DMA,
            pltpu.SemaphoreType.DMA,
            pltpu.SemaphoreType.DMA,
            pltpu.SemaphoreType.DMA,
            pltpu.SemaphoreType.DMA,
        ],
    )(functools.partial(_agg_body, n_ch))


def _pre_tc(h_ref, w_ref, deg_ref, y_ref):
    dis = lax.rsqrt(deg_ref[...])
    xw = jnp.dot(h_ref[...], w_ref[...], preferred_element_type=jnp.float32)
    y = xw * dis
    y_ref[0:N_NODES, :] = y[:, 0:DH]
    y_ref[N_ACC:N_ACC + N_NODES, :] = y[:, DH:]


def _mid_tc(t_ref, deg_ref, b_ref, g_ref, bt_ref, w_ref, y_ref):
    dis = lax.rsqrt(deg_ref[...])
    tmp = jnp.concatenate(
        [t_ref[0:N_NODES, :], t_ref[N_ACC:N_ACC + N_NODES, :]], axis=1)
    a = tmp * dis + b_ref[...]
    mu = jnp.mean(a, axis=0, keepdims=True)
    dmu = a - mu
    var = jnp.mean(dmu * dmu, axis=0, keepdims=True)
    hb = dmu * lax.rsqrt(var + EPS) * g_ref[...] + bt_ref[...]
    hl = jnp.where(hb >= 0, hb, SLOPE * hb)
    xw = jnp.dot(hl, w_ref[...], preferred_element_type=jnp.float32)
    y = xw * dis
    y_ref[0:N_NODES, :] = y[:, 0:DH]
    y_ref[N_ACC:N_ACC + N_NODES, :] = y[:, DH:]


def _post_tc(t_ref, deg_ref, b_ref, o_ref):
    dis = lax.rsqrt(deg_ref[...])
    tmp = jnp.concatenate(
        [t_ref[0:N_NODES, :], t_ref[N_ACC:N_ACC + N_NODES, :]], axis=1)
    o_ref[...] = tmp * dis + b_ref[...]


def _pre_call(h8, w8, deg):
    return pl.pallas_call(
        _pre_tc,
        out_shape=jax.ShapeDtypeStruct((2 * N_ACC, DH), jnp.float32),
    )(h8, w8, deg)


def _mid_call(t, deg, b, g, bt, w):
    return pl.pallas_call(
        _mid_tc,
        out_shape=jax.ShapeDtypeStruct((2 * N_ACC, DH), jnp.float32),
    )(t, deg, b, g, bt, w)


def _post_call(t, deg, b):
    return pl.pallas_call(
        _post_tc,
        out_shape=jax.ShapeDtypeStruct((N_NODES, D), jnp.float32),
    )(t, deg, b)


def kernel(h, edge_index, edge_weight, W1, b1, Ws, bs, Wl, bl, gamma, beta):
    src = edge_index[0].astype(jnp.int32)
    dst = edge_index[1].astype(jnp.int32)
    ew = edge_weight.astype(jnp.float32)
    e = src.shape[0]
    per = NSUB * CHUNK
    n_ch = -(-e // per)
    n_ch += n_ch % 2
    pad = n_ch * per - e
    src_p = jnp.pad(src, (0, pad))
    dst_p = jnp.pad(dst, (0, pad))
    ew_p = jnp.pad(ew, (0, pad))
    src2 = jnp.concatenate([src_p, src_p + N_ACC])
    dst3 = dst_p.reshape(NSUB, n_ch, CHUNK)

    degp = _make_deg_call(n_ch)(dst_p, ew_p)
    deg = (degp[:N_NODES] + degp[DEG_PAD:DEG_PAD + N_NODES] + 1.0)
    deg = deg.reshape(N_NODES, 1)

    h8 = jnp.pad(h, ((0, 0), (0, 8 - h.shape[1])))
    w18 = jnp.pad(W1, ((0, 8 - W1.shape[0]), (0, 0)))
    agg = _make_agg_call(n_ch)

    y = _pre_call(h8, w18, deg)
    biases = [b1] + [bs[i] for i in range(6)]
    w_next = [Ws[i] for i in range(6)] + [Wl]
    for i in range(7):
        t = agg(y, src2, dst3, ew_p)
        y = _mid_call(t, deg, biases[i].reshape(1, D),
                      gamma[i].reshape(1, D), beta[i].reshape(1, D), w_next[i])
    t = agg(y, src2, dst3, ew_p)
    return _post_call(t, deg, bl.reshape(1, D))


# speedup vs baseline: 1.1393x; 1.1393x over previous
import functools

import jax
import jax.numpy as jnp
from jax import lax
from jax.experimental import pallas as pl
from jax.experimental.pallas import tpu as pltpu
from jax.experimental.pallas import tpu_sc as plsc

N_NODES = 10000
D = 256
DH = 128
EPS = 1e-5
SLOPE = 0.01
NSUB = 16
CHUNK = 128
N_ACC = 10112
ROWS_PER_TILE = N_ACC // NSUB
DEG_SLAB = 640
DEG_PAD = NSUB * DEG_SLAB

_MESH = plsc.VectorSubcoreMesh(core_axis_name="c", subcore_axis_name="s")


def _deg_body(n_ch, dst_hbm, ew_hbm, out_hbm, didx_v, ew_v, zero_v, acc_sh,
              sem):
    del sem
    c = lax.axis_index("c")
    s = lax.axis_index("s")
    for k in range(DEG_SLAB // 16):
        zero_v[pl.ds(k * 16, 16)] = jnp.zeros((16,), jnp.float32)
    pltpu.sync_copy(zero_v, acc_sh.at[pl.ds(s * DEG_SLAB, DEG_SLAB)])
    plsc.subcore_barrier()
    half = (n_ch + 1) // 2
    lo = jnp.where(c == 0, 0, half)
    hi = jnp.where(c == 0, half, n_ch)

    def body(j, carry):
        off = pl.multiple_of((s * n_ch + j) * CHUNK, CHUNK)
        pltpu.sync_copy(dst_hbm.at[pl.ds(off, CHUNK)], didx_v)
        pltpu.sync_copy(ew_hbm.at[pl.ds(off, CHUNK)], ew_v)
        pltpu.sync_copy(ew_v, acc_sh.at[didx_v], add=True)
        return carry

    lax.fori_loop(lo, hi, body, 0)
    plsc.subcore_barrier()
    out_off = pl.multiple_of(c * DEG_PAD + s * DEG_SLAB, DEG_SLAB)
    pltpu.sync_copy(acc_sh.at[pl.ds(s * DEG_SLAB, DEG_SLAB)],
                    out_hbm.at[pl.ds(out_off, DEG_SLAB)])


def _make_deg_call(n_ch):
    return functools.partial(
        pl.kernel,
        mesh=_MESH,
        out_type=jax.ShapeDtypeStruct((2 * DEG_PAD,), jnp.float32),
        scratch_types=[
            pltpu.VMEM((CHUNK,), jnp.int32),
            pltpu.VMEM((CHUNK,), jnp.float32),
            pltpu.VMEM((DEG_SLAB,), jnp.float32),
            pltpu.VMEM_SHARED((DEG_PAD,), jnp.float32),
            pltpu.SemaphoreType.DMA,
        ],
    )(functools.partial(_deg_body, n_ch))


def _agg_body(n_ch, y_hbm, src_hbm, dst_hbm, ew_hbm, out_hbm,
              dst_all, sidx0_v, sidx1_v, ew0_v, ew1_v, rows0_v, rows1_v,
              acc_sh, semg0, semg1, semw0, semw1, sems0, sems1):
    c = lax.axis_index("c")
    s = lax.axis_index("s")
    y_base = c * N_ACC
    row0 = s * ROWS_PER_TILE
    slab0 = pl.multiple_of(y_base + row0, 8)
    rows = (rows0_v, rows1_v)
    ews = (ew0_v, ew1_v)
    sidx = (sidx0_v, sidx1_v)
    semg = (semg0, semg1)
    semw = (semw0, semw1)
    sems = (sems0, sems1)
    sbase = s * n_ch * CHUNK
    n_pairs = n_ch // 2
    _dn = lax.GatherDimensionNumbers(offset_dims=(), collapsed_slice_dims=(0,),
                                     start_index_map=(0,))

    pltpu.sync_copy(dst_hbm.at[s], dst_all)
    pltpu.sync_copy(y_hbm.at[pl.ds(slab0, ROWS_PER_TILE)],
                    acc_sh.at[pl.ds(row0, ROWS_PER_TILE)])
    plsc.subcore_barrier()

    def loads(j, slot):
        soff = pl.multiple_of(sbase + j * CHUNK, CHUNK)
        pltpu.make_async_copy(src_hbm.at[pl.ds(soff, CHUNK)],
                              sidx[slot], sems[slot]).start()
        pltpu.make_async_copy(ew_hbm.at[pl.ds(soff, CHUNK)],
                              ews[slot], semw[slot]).start()

    def gather(j, slot):
        soff = pl.multiple_of(sbase + j * CHUNK, CHUNK)
        pltpu.make_async_copy(src_hbm.at[pl.ds(soff, CHUNK)],
                              sidx[slot], sems[slot]).wait()
        for k in range(CHUNK // 16):
            sidx[slot][pl.ds(k * 16, 16)] = (
                sidx[slot][pl.ds(k * 16, 16)] + y_base)
        pltpu.make_async_copy(y_hbm.at[sidx[slot]], rows[slot],
                              semg[slot]).start()

    def consume(j, slot):
        pltpu.make_async_copy(y_hbm.at[sidx[slot]], rows[slot],
                              semg[slot]).wait()
        soff = pl.multiple_of(sbase + j * CHUNK, CHUNK)
        pltpu.make_async_copy(ew_hbm.at[pl.ds(soff, CHUNK)],
                              ews[slot], semw[slot]).wait()

        def grp(g, inner):
            g16 = pl.multiple_of(g * 16, 16)
            ew16 = ews[slot][pl.ds(g16, 16)]
            for l in range(16):
                w16 = lax.gather(ew16, jnp.full((16, 1), l, jnp.int32), _dn,
                                 slice_sizes=(1,),
                                 mode=lax.GatherScatterMode.PROMISE_IN_BOUNDS)
                e = g16 + l
                for k in range(DH // 16):
                    rows[slot][e, pl.ds(k * 16, 16)] = (
                        rows[slot][e, pl.ds(k * 16, 16)] * w16)
            return inner

        lax.fori_loop(0, CHUNK // 16, grp, 0)
        pltpu.sync_copy(rows[slot], acc_sh.at[dst_all.at[j]], add=True)

    loads(0, 0)
    gather(0, 0)
    loads(1, 1)

    def body(p, carry):
        j0 = 2 * p
        j1 = j0 + 1
        more = p + 1 < n_pairs
        gather(j1, 1)
        consume(j0, 0)

        @pl.when(more)
        def _n0():
            loads(j0 + 2, 0)
            gather(j0 + 2, 0)

        consume(j1, 1)

        @pl.when(more)
        def _n1():
            loads(j1 + 2, 1)

        return carry

    lax.fori_loop(0, n_pairs, body, 0)
    plsc.subcore_barrier()
    pltpu.sync_copy(acc_sh.at[pl.ds(row0, ROWS_PER_TILE)],
                    out_hbm.at[pl.ds(slab0, ROWS_PER_TILE)])


def _make_agg_call(n_ch):
    return functools.partial(
        pl.kernel,
        mesh=_MESH,
        out_type=jax.ShapeDtypeStruct((2 * N_ACC, DH), jnp.float32),
        scratch_types=[
            pltpu.VMEM((n_ch, CHUNK), jnp.int32),
            pltpu.VMEM((CHUNK,), jnp.int32),
            pltpu.VMEM((CHUNK,), jnp.int32),
            pltpu.VMEM((CHUNK,), jnp.float32),
            pltpu.VMEM((CHUNK,), jnp.float32),
            pltpu.VMEM((CHUNK, DH), jnp.float32),
            pltpu.VMEM((CHUNK, DH), jnp.float32),
            pltpu.VMEM_SHARED((N_ACC, DH), jnp.float32),
            pltpu.SemaphoreType.DMA,
            pltpu.SemaphoreType.DMA,
            pltpu.SemaphoreType.DMA,
            pltpu.SemaphoreType.DMA,
            pltpu.SemaphoreType.DMA,
            pltpu.SemaphoreType.DMA,
        ],
    )(functools.partial(_agg_body, n_ch))


def _pre_tc(h_ref, w_ref, deg_ref, y_ref):
    dis = lax.rsqrt(deg_ref[...])
    xw = jnp.dot(h_ref[...], w_ref[...], preferred_element_type=jnp.float32)
    y = xw * dis
    y_ref[0:N_NODES, :] = y[:, 0:DH]
    y_ref[N_ACC:N_ACC + N_NODES, :] = y[:, DH:]


def _mid_tc(t_ref, deg_ref, b_ref, g_ref, bt_ref, w_ref, y_ref):
    dis = lax.rsqrt(deg_ref[...])
    tmp = jnp.concatenate(
        [t_ref[0:N_NODES, :], t_ref[N_ACC:N_ACC + N_NODES, :]], axis=1)
    a = tmp * dis + b_ref[...]
    mu = jnp.mean(a, axis=0, keepdims=True)
    dmu = a - mu
    var = jnp.mean(dmu * dmu, axis=0, keepdims=True)
    hb = dmu * lax.rsqrt(var + EPS) * g_ref[...] + bt_ref[...]
    hl = jnp.where(hb >= 0, hb, SLOPE * hb)
    xw = jnp.dot(hl, w_ref[...], preferred_element_type=jnp.float32)
    y = xw * dis
    y_ref[0:N_NODES, :] = y[:, 0:DH]
    y_ref[N_ACC:N_ACC + N_NODES, :] = y[:, DH:]


def _post_tc(t_ref, deg_ref, b_ref, o_ref):
    dis = lax.rsqrt(deg_ref[...])
    tmp = jnp.concatenate(
        [t_ref[0:N_NODES, :], t_ref[N_ACC:N_ACC + N_NODES, :]], axis=1)
    o_ref[...] = tmp * dis + b_ref[...]


def _pre_call(h8, w8, deg):
    return pl.pallas_call(
        _pre_tc,
        out_shape=jax.ShapeDtypeStruct((2 * N_ACC, DH), jnp.float32),
    )(h8, w8, deg)


def _mid_call(t, deg, b, g, bt, w):
    return pl.pallas_call(
        _mid_tc,
        out_shape=jax.ShapeDtypeStruct((2 * N_ACC, DH), jnp.float32),
    )(t, deg, b, g, bt, w)


def _post_call(t, deg, b):
    return pl.pallas_call(
        _post_tc,
        out_shape=jax.ShapeDtypeStruct((N_NODES, D), jnp.float32),
    )(t, deg, b)


def kernel(h, edge_index, edge_weight, W1, b1, Ws, bs, Wl, bl, gamma, beta):
    src = edge_index[0].astype(jnp.int32)
    dst = edge_index[1].astype(jnp.int32)
    ew = edge_weight.astype(jnp.float32)
    e = src.shape[0]
    per = NSUB * CHUNK
    n_ch = -(-e // per)
    n_ch += n_ch % 2
    pad = n_ch * per - e
    src_p = jnp.pad(src, (0, pad))
    dst_p = jnp.pad(dst, (0, pad))
    ew_p = jnp.pad(ew, (0, pad))
    src3 = src_p.reshape(NSUB, n_ch, CHUNK)
    dst3 = dst_p.reshape(NSUB, n_ch, CHUNK)

    degp = _make_deg_call(n_ch)(dst_p, ew_p)
    deg = (degp[:N_NODES] + degp[DEG_PAD:DEG_PAD + N_NODES] + 1.0)
    deg = deg.reshape(N_NODES, 1)

    h8 = jnp.pad(h, ((0, 0), (0, 8 - h.shape[1])))
    w18 = jnp.pad(W1, ((0, 8 - W1.shape[0]), (0, 0)))
    agg = _make_agg_call(n_ch)

    y = _pre_call(h8, w18, deg)
    biases = [b1] + [bs[i] for i in range(6)]
    w_next = [Ws[i] for i in range(6)] + [Wl]
    for i in range(7):
        t = agg(y, src_p, dst3, ew_p)
        y = _mid_call(t, deg, biases[i].reshape(1, D),
                      gamma[i].reshape(1, D), beta[i].reshape(1, D), w_next[i])
    t = agg(y, src_p, dst3, ew_p)
    return _post_call(t, deg, bl.reshape(1, D))


# speedup vs baseline: 1.1936x; 1.0477x over previous
import functools

import jax
import jax.numpy as jnp
from jax import lax
from jax.experimental import pallas as pl
from jax.experimental.pallas import tpu as pltpu
from jax.experimental.pallas import tpu_sc as plsc

N_NODES = 10000
D = 256
DH = 128
EPS = 1e-5
SLOPE = 0.01
NSUB = 16
CHUNK = 128
N_ACC = 10112
ROWS_PER_TILE = N_ACC // NSUB
DEG_SLAB = 640
DEG_PAD = NSUB * DEG_SLAB

_MESH = plsc.VectorSubcoreMesh(core_axis_name="c", subcore_axis_name="s")


def _deg_body(n_ch, dst_hbm, ew_hbm, out_hbm, didx_v, ew_v, zero_v, acc_sh,
              sem):
    del sem
    c = lax.axis_index("c")
    s = lax.axis_index("s")
    for k in range(DEG_SLAB // 16):
        zero_v[pl.ds(k * 16, 16)] = jnp.zeros((16,), jnp.float32)
    pltpu.sync_copy(zero_v, acc_sh.at[pl.ds(s * DEG_SLAB, DEG_SLAB)])
    plsc.subcore_barrier()
    half = (n_ch + 1) // 2
    lo = jnp.where(c == 0, 0, half)
    hi = jnp.where(c == 0, half, n_ch)

    def body(j, carry):
        off = pl.multiple_of((s * n_ch + j) * CHUNK, CHUNK)
        pltpu.sync_copy(dst_hbm.at[pl.ds(off, CHUNK)], didx_v)
        pltpu.sync_copy(ew_hbm.at[pl.ds(off, CHUNK)], ew_v)
        pltpu.sync_copy(ew_v, acc_sh.at[didx_v], add=True)
        return carry

    lax.fori_loop(lo, hi, body, 0)
    plsc.subcore_barrier()
    out_off = pl.multiple_of(c * DEG_PAD + s * DEG_SLAB, DEG_SLAB)
    pltpu.sync_copy(acc_sh.at[pl.ds(s * DEG_SLAB, DEG_SLAB)],
                    out_hbm.at[pl.ds(out_off, DEG_SLAB)])


def _make_deg_call(n_ch):
    return functools.partial(
        pl.kernel,
        mesh=_MESH,
        out_type=jax.ShapeDtypeStruct((2 * DEG_PAD,), jnp.float32),
        scratch_types=[
            pltpu.VMEM((CHUNK,), jnp.int32),
            pltpu.VMEM((CHUNK,), jnp.float32),
            pltpu.VMEM((DEG_SLAB,), jnp.float32),
            pltpu.VMEM_SHARED((DEG_PAD,), jnp.float32),
            pltpu.SemaphoreType.DMA,
        ],
    )(functools.partial(_deg_body, n_ch))


def _agg_body(n_ch, y_hbm, src_hbm, dst_hbm, ew_hbm, out_hbm,
              dst_all, sidx0_v, sidx1_v, ew0_v, ew1_v, rows0_v, rows1_v,
              acc_sh, semg0, semg1, semw0, semw1, sems0, sems1):
    c = lax.axis_index("c")
    s = lax.axis_index("s")
    y_base = c * N_ACC
    row0 = s * ROWS_PER_TILE
    slab0 = pl.multiple_of(y_base + row0, 8)
    rows = (rows0_v, rows1_v)
    ews = (ew0_v, ew1_v)
    sidx = (sidx0_v, sidx1_v)
    semg = (semg0, semg1)
    semw = (semw0, semw1)
    sems = (sems0, sems1)
    sbase = s * n_ch * CHUNK
    n_pairs = n_ch // 2
    _dn = lax.GatherDimensionNumbers(offset_dims=(), collapsed_slice_dims=(0,),
                                     start_index_map=(0,))

    pltpu.sync_copy(dst_hbm.at[s], dst_all)
    pltpu.sync_copy(y_hbm.at[pl.ds(slab0, ROWS_PER_TILE)],
                    acc_sh.at[pl.ds(row0, ROWS_PER_TILE)])
    plsc.subcore_barrier()

    def loads(j, slot):
        soff = pl.multiple_of(sbase + j * CHUNK, CHUNK)
        pltpu.make_async_copy(src_hbm.at[pl.ds(soff, CHUNK)],
                              sidx[slot], sems[slot]).start()
        pltpu.make_async_copy(ew_hbm.at[pl.ds(soff, CHUNK)],
                              ews[slot], semw[slot]).start()

    def gather(j, slot):
        soff = pl.multiple_of(sbase + j * CHUNK, CHUNK)
        pltpu.make_async_copy(src_hbm.at[pl.ds(soff, CHUNK)],
                              sidx[slot], sems[slot]).wait()
        for k in range(CHUNK // 16):
            sidx[slot][pl.ds(k * 16, 16)] = (
                sidx[slot][pl.ds(k * 16, 16)] + y_base)
        pltpu.make_async_copy(y_hbm.at[sidx[slot]], rows[slot],
                              semg[slot]).start()

    def consume(j, slot, more):
        pltpu.make_async_copy(y_hbm.at[sidx[slot]], rows[slot],
                              semg[slot]).wait()

        @pl.when(more)
        def _ps():
            soff2 = pl.multiple_of(sbase + (j + 2) * CHUNK, CHUNK)
            pltpu.make_async_copy(src_hbm.at[pl.ds(soff2, CHUNK)],
                                  sidx[slot], sems[slot]).start()

        soff = pl.multiple_of(sbase + j * CHUNK, CHUNK)
        pltpu.make_async_copy(ew_hbm.at[pl.ds(soff, CHUNK)],
                              ews[slot], semw[slot]).wait()

        def grp(g, inner):
            g16 = pl.multiple_of(g * 16, 16)
            ew16 = ews[slot][pl.ds(g16, 16)]
            for l in range(16):
                w16 = lax.gather(ew16, jnp.full((16, 1), l, jnp.int32), _dn,
                                 slice_sizes=(1,),
                                 mode=lax.GatherScatterMode.PROMISE_IN_BOUNDS)
                e = g16 + l
                for k in range(DH // 16):
                    rows[slot][e, pl.ds(k * 16, 16)] = (
                        rows[slot][e, pl.ds(k * 16, 16)] * w16)
            return inner

        lax.fori_loop(0, CHUNK // 16, grp, 0)

        @pl.when(more)
        def _pw():
            soff2 = pl.multiple_of(sbase + (j + 2) * CHUNK, CHUNK)
            pltpu.make_async_copy(ew_hbm.at[pl.ds(soff2, CHUNK)],
                                  ews[slot], semw[slot]).start()

        pltpu.sync_copy(rows[slot], acc_sh.at[dst_all.at[j]], add=True)

    loads(0, 0)
    gather(0, 0)
    loads(1, 1)

    def body(p, carry):
        j0 = 2 * p
        j1 = j0 + 1
        more = p + 1 < n_pairs
        gather(j1, 1)
        consume(j0, 0, more)

        @pl.when(more)
        def _n0():
            gather(j0 + 2, 0)

        consume(j1, 1, more)
        return carry

    lax.fori_loop(0, n_pairs, body, 0)
    plsc.subcore_barrier()
    pltpu.sync_copy(acc_sh.at[pl.ds(row0, ROWS_PER_TILE)],
                    out_hbm.at[pl.ds(slab0, ROWS_PER_TILE)])


def _make_agg_call(n_ch):
    return functools.partial(
        pl.kernel,
        mesh=_MESH,
        out_type=jax.ShapeDtypeStruct((2 * N_ACC, DH), jnp.float32),
        scratch_types=[
            pltpu.VMEM((n_ch, CHUNK), jnp.int32),
            pltpu.VMEM((CHUNK,), jnp.int32),
            pltpu.VMEM((CHUNK,), jnp.int32),
            pltpu.VMEM((CHUNK,), jnp.float32),
            pltpu.VMEM((CHUNK,), jnp.float32),
            pltpu.VMEM((CHUNK, DH), jnp.float32),
            pltpu.VMEM((CHUNK, DH), jnp.float32),
            pltpu.VMEM_SHARED((N_ACC, DH), jnp.float32),
            pltpu.SemaphoreType.DMA,
            pltpu.SemaphoreType.DMA,
            pltpu.SemaphoreType.DMA,
            pltpu.SemaphoreType.DMA,
            pltpu.SemaphoreType.DMA,
            pltpu.SemaphoreType.DMA,
        ],
    )(functools.partial(_agg_body, n_ch))


def _pre_tc(h_ref, w_ref, deg_ref, y_ref):
    dis = lax.rsqrt(deg_ref[...])
    xw = jnp.dot(h_ref[...], w_ref[...], preferred_element_type=jnp.float32)
    y = xw * dis
    y_ref[0:N_NODES, :] = y[:, 0:DH]
    y_ref[N_ACC:N_ACC + N_NODES, :] = y[:, DH:]


def _mid_tc(t_ref, deg_ref, b_ref, g_ref, bt_ref, w_ref, y_ref):
    dis = lax.rsqrt(deg_ref[...])
    tmp = jnp.concatenate(
        [t_ref[0:N_NODES, :], t_ref[N_ACC:N_ACC + N_NODES, :]], axis=1)
    a = tmp * dis + b_ref[...]
    mu = jnp.mean(a, axis=0, keepdims=True)
    dmu = a - mu
    var = jnp.mean(dmu * dmu, axis=0, keepdims=True)
    hb = dmu * lax.rsqrt(var + EPS) * g_ref[...] + bt_ref[...]
    hl = jnp.where(hb >= 0, hb, SLOPE * hb)
    xw = jnp.dot(hl, w_ref[...], preferred_element_type=jnp.float32)
    y = xw * dis
    y_ref[0:N_NODES, :] = y[:, 0:DH]
    y_ref[N_ACC:N_ACC + N_NODES, :] = y[:, DH:]


def _post_tc(t_ref, deg_ref, b_ref, o_ref):
    dis = lax.rsqrt(deg_ref[...])
    tmp = jnp.concatenate(
        [t_ref[0:N_NODES, :], t_ref[N_ACC:N_ACC + N_NODES, :]], axis=1)
    o_ref[...] = tmp * dis + b_ref[...]


def _pre_call(h8, w8, deg):
    return pl.pallas_call(
        _pre_tc,
        out_shape=jax.ShapeDtypeStruct((2 * N_ACC, DH), jnp.float32),
    )(h8, w8, deg)


def _mid_call(t, deg, b, g, bt, w):
    return pl.pallas_call(
        _mid_tc,
        out_shape=jax.ShapeDtypeStruct((2 * N_ACC, DH), jnp.float32),
    )(t, deg, b, g, bt, w)


def _post_call(t, deg, b):
    return pl.pallas_call(
        _post_tc,
        out_shape=jax.ShapeDtypeStruct((N_NODES, D), jnp.float32),
    )(t, deg, b)


def kernel(h, edge_index, edge_weight, W1, b1, Ws, bs, Wl, bl, gamma, beta):
    src = edge_index[0].astype(jnp.int32)
    dst = edge_index[1].astype(jnp.int32)
    ew = edge_weight.astype(jnp.float32)
    e = src.shape[0]
    per = NSUB * CHUNK
    n_ch = -(-e // per)
    n_ch += n_ch % 2
    pad = n_ch * per - e
    src_p = jnp.pad(src, (0, pad))
    dst_p = jnp.pad(dst, (0, pad))
    ew_p = jnp.pad(ew, (0, pad))
    src3 = src_p.reshape(NSUB, n_ch, CHUNK)
    dst3 = dst_p.reshape(NSUB, n_ch, CHUNK)

    degp = _make_deg_call(n_ch)(dst_p, ew_p)
    deg = (degp[:N_NODES] + degp[DEG_PAD:DEG_PAD + N_NODES] + 1.0)
    deg = deg.reshape(N_NODES, 1)

    h8 = jnp.pad(h, ((0, 0), (0, 8 - h.shape[1])))
    w18 = jnp.pad(W1, ((0, 8 - W1.shape[0]), (0, 0)))
    agg = _make_agg_call(n_ch)

    y = _pre_call(h8, w18, deg)
    biases = [b1] + [bs[i] for i in range(6)]
    w_next = [Ws[i] for i in range(6)] + [Wl]
    for i in range(7):
        t = agg(y, src_p, dst3, ew_p)
        y = _mid_call(t, deg, biases[i].reshape(1, D),
                      gamma[i].reshape(1, D), beta[i].reshape(1, D), w_next[i])
    t = agg(y, src_p, dst3, ew_p)
    return _post_call(t, deg, bl.reshape(1, D))
